# Initial kernel scaffold; baseline (speedup 1.0000x reference)
import jax
import jax.numpy as jnp
from jax.experimental import pallas as pl


def kernel(cat_fea_sex, cat_fea_level_id, iter_fea_shop_id, iter_fea_cate, iter_fea_floor, candidate_shop_id, candidate_cate, embed_sex, embed_level, embed_shop, embed_cate, embed_floor, W1u, b1u, W1i, b1i, W2, b2, W3, b3, gamma, beta):
    raise NotImplementedError("write your pallas kernel here")


# speedup vs baseline: 2.8112x; 2.8112x over previous
import functools

import jax
import jax.numpy as jnp
from jax import lax
from jax.experimental import pallas as pl
from jax.experimental.pallas import tpu as pltpu
from jax.experimental.pallas import tpu_sc as plsc

B = 4096
L = 50
D = 64
NC, NS = 2, 16
NW = NC * NS
PB = B // NW
CB = 16
NCHUNK = PB // CB
RPC = CB * L
GSZ = 80
NG = RPC // GSZ


def _sc_body(sex_i, lvl_i, shop_i, cate_i, floor_i, cshop_i, ccate_i,
             t_sex, t_lvl, t_shop, t_cate, t_floor,
             o_sex, o_lvl, o_shop, o_cate, o_floor, o_cshop, o_ccate,
             idx_s, rows_s, idx_p, rows_p, pool_s, sem):
    wid = lax.axis_index("s") * NC + lax.axis_index("c")
    base = wid * PB

    for idx_hbm, table, out in ((sex_i, t_sex, o_sex),
                                (lvl_i, t_lvl, o_lvl),
                                (cshop_i, t_shop, o_cshop),
                                (ccate_i, t_cate, o_ccate)):
        pltpu.sync_copy(idx_hbm.at[pl.ds(base, PB)], idx_s)
        pltpu.async_copy(table.at[idx_s], rows_s, sem).wait()
        pltpu.sync_copy(rows_s, out.at[pl.ds(base, PB)])

    def make_chunk_body(idx_hbm, table, out):
        def chunk_body(c, carry):
            start = (base + c * CB) * L
            pltpu.sync_copy(idx_hbm.at[pl.ds(start, RPC)], idx_p)
            cps = [pltpu.async_copy(table.at[idx_p.at[pl.ds(g * GSZ, GSZ)]],
                                    rows_p.at[pl.ds(g * GSZ, GSZ)], sem)
                   for g in range(NG)]
            for cp in cps:
                cp.wait()
            for b in range(CB):
                rb = b * L

                def jbody(j, accs):
                    return tuple(a + rows_p[rb + j, pl.ds(16 * k, 16)]
                                 for k, a in enumerate(accs))

                z = jnp.zeros((16,), jnp.float32)
                accs = lax.fori_loop(0, L, jbody, (z, z, z, z))
                for k in range(4):
                    pool_s[b, pl.ds(16 * k, 16)] = accs[k]
            pltpu.sync_copy(pool_s, out.at[pl.ds(base + c * CB, CB)])
            return carry
        return chunk_body

    for idx_hbm, table, out in ((shop_i, t_shop, o_shop),
                                (cate_i, t_cate, o_cate),
                                (floor_i, t_floor, o_floor)):
        lax.fori_loop(0, NCHUNK, make_chunk_body(idx_hbm, table, out), 0)


@functools.partial(
    pl.kernel,
    out_type=[jax.ShapeDtypeStruct((B, D), jnp.float32)] * 7,
    mesh=plsc.VectorSubcoreMesh(core_axis_name="c", subcore_axis_name="s",
                                num_cores=NC, num_subcores=NS),
    scratch_types=[
        pltpu.VMEM((PB,), jnp.int32),
        pltpu.VMEM((PB, D), jnp.float32),
        pltpu.VMEM((RPC,), jnp.int32),
        pltpu.VMEM((RPC, D), jnp.float32),
        pltpu.VMEM((CB, D), jnp.float32),
        pltpu.SemaphoreType.DMA,
    ],
    compiler_params=pltpu.CompilerParams(use_tc_tiling_on_sc=False),
)
def _sc_gather(*refs):
    _sc_body(*refs)


def _tc_mlp(es, el, eshop, ecate, efloor, ish, ica,
            w1u, b1u, w1i, b1i, w2, b2, w3, b3, g, be, out):
    dot = lambda x, w: lax.dot_general(
        x, w, (((1,), (1,)), ((), ())), preferred_element_type=jnp.float32)

    def bn(x):
        m = jnp.mean(x, axis=0, keepdims=True)
        v = jnp.mean((x - m) ** 2, axis=0, keepdims=True)
        return g[...] * (x - m) / jnp.sqrt(v + 1e-5) + be[...]

    w1u_ = w1u[...]
    u = (dot(es[...], w1u_[:, 0:D]) + dot(el[...], w1u_[:, D:2 * D])
         + dot(eshop[...], w1u_[:, 2 * D:3 * D])
         + dot(ecate[...], w1u_[:, 3 * D:4 * D])
         + dot(efloor[...], w1u_[:, 4 * D:5 * D]))
    u = jnp.tanh(u + b1u[...])
    u = bn(u)
    u = jnp.tanh(dot(u, w2[...]) + b2[...])
    u = bn(u)
    u = jnp.tanh(dot(u, w3[...]) + b3[...])

    w1i_ = w1i[...]
    it = dot(ish[...], w1i_[:, 0:D]) + dot(ica[...], w1i_[:, D:2 * D])
    it = jnp.tanh(it + b1i[...])
    it = jnp.tanh(dot(it, w2[...]) + b2[...])
    it = jnp.tanh(dot(it, w3[...]) + b3[...])

    eps = 1e-8
    nu = jnp.maximum(jnp.sqrt(jnp.sum(u * u, axis=1, keepdims=True)), eps)
    ni = jnp.maximum(jnp.sqrt(jnp.sum(it * it, axis=1, keepdims=True)), eps)
    out[...] = jnp.sum(u * it, axis=1, keepdims=True) / (nu * ni)


def kernel(cat_fea_sex, cat_fea_level_id, iter_fea_shop_id, iter_fea_cate,
           iter_fea_floor, candidate_shop_id, candidate_cate,
           embed_sex, embed_level, embed_shop, embed_cate, embed_floor,
           W1u, b1u, W1i, b1i, W2, b2, W3, b3, gamma, beta):
    es, el, eshop, ecate, efloor, ish, ica = _sc_gather(
        cat_fea_sex, cat_fea_level_id,
        iter_fea_shop_id.reshape(-1), iter_fea_cate.reshape(-1),
        iter_fea_floor.reshape(-1),
        candidate_shop_id, candidate_cate,
        embed_sex, embed_level, embed_shop, embed_cate, embed_floor)

    out = pl.pallas_call(
        _tc_mlp,
        out_shape=jax.ShapeDtypeStruct((B, 1), jnp.float32),
    )(es, el, eshop, ecate, efloor, ish, ica,
      W1u, b1u.reshape(1, -1), W1i, b1i.reshape(1, -1),
      W2, b2.reshape(1, -1), W3, b3.reshape(1, -1),
      gamma.reshape(1, -1), beta.reshape(1, -1))
    return out.reshape(B)
